# Initial kernel scaffold; baseline (speedup 1.0000x reference)
import jax
import jax.numpy as jnp
from jax.experimental import pallas as pl


def kernel(x, layer_idx, W, b):
    raise NotImplementedError("write your pallas kernel here")


# speedup vs baseline: 278.0517x; 278.0517x over previous
import functools

import jax
import jax.numpy as jnp
from jax.experimental import pallas as pl


def _masked_body(oh_ref, x_ref, w_ref, b_ref, o_ref):
    e = pl.program_id(1)
    n_e = pl.num_programs(1)
    sel = (jax.lax.broadcasted_iota(jnp.int32, (oh_ref.shape[1], 1), 0) == e)
    member = jnp.dot(oh_ref[...], sel.astype(jnp.bfloat16),
                     preferred_element_type=jnp.float32)
    xm = x_ref[...] * member.astype(jnp.bfloat16)
    mm = jnp.dot(xm, w_ref[0], preferred_element_type=jnp.float32)
    contrib = mm + member * b_ref[0]

    @pl.when(e == 0)
    def _():
        o_ref[...] = contrib

    @pl.when(e != 0)
    def _():
        o_ref[...] += contrib


def kernel(x, layer_idx, W, b):
    N, D_IN = x.shape
    E, _, D_OUT = W.shape
    TBLK = min(1024, N)
    T = N // TBLK

    x_bf = x.astype(jnp.bfloat16)
    W_bf = W.astype(jnp.bfloat16)
    onehot = (layer_idx.astype(jnp.int32)[:, None]
              == jnp.arange(E, dtype=jnp.int32)[None, :]).astype(jnp.bfloat16)

    grid = (T, E)
    return pl.pallas_call(
        _masked_body,
        grid=grid,
        in_specs=[
            pl.BlockSpec((TBLK, E), lambda t, e: (t, 0)),
            pl.BlockSpec((TBLK, D_IN), lambda t, e: (t, 0)),
            pl.BlockSpec((1, D_IN, D_OUT), lambda t, e: (e, 0, 0)),
            pl.BlockSpec((1, 1, D_OUT), lambda t, e: (e, 0, 0)),
        ],
        out_specs=pl.BlockSpec((TBLK, D_OUT), lambda t, e: (t, 0)),
        out_shape=jax.ShapeDtypeStruct((N, D_OUT), jnp.float32),
    )(onehot, x_bf, W_bf, b.reshape(E, 1, D_OUT))


# speedup vs baseline: 521.0526x; 1.8739x over previous
import functools

import jax
import jax.numpy as jnp
from jax import lax
from jax.experimental import pallas as pl
from jax.experimental.pallas import tpu as pltpu
from jax.experimental.pallas import tpu_sc as plsc

N_TOK = 8192
N_EXP = 8
D_IN = 2048
D_OUT = 2048

NC = 2
NS = 16
NW = NC * NS
LANES = 16

TPT = N_TOK // NW
RVECS = TPT // LANES

RPW = N_TOK // NW
CHUNK = 16
NCHUNK = RPW // CHUNK

TBLK = 512
T_BLOCKS = N_TOK // TBLK
KMAX = N_EXP


def _wid():
    return lax.axis_index("c") * NS + lax.axis_index("s")


def _count_body(idx_hbm, lrank_hbm, cnt_hbm, idx_v, lrank_v, cnt_v, run_s):
    wid = _wid()
    base = pl.multiple_of(wid * TPT, 8)
    pltpu.sync_copy(idx_hbm.at[pl.ds(base, TPT)], idx_v)
    iota = lax.broadcasted_iota(jnp.int32, (LANES,), 0)

    for e in range(N_EXP):
        run_s[e] = 0

    def pass1(v, _):
        ev = idx_v[pl.ds(v * LANES, LANES)]
        lr = jnp.zeros((LANES,), jnp.int32)
        for j in range(LANES):
            e = ev[j]
            r = run_s[e]
            run_s[e] = r + 1
            lr = jnp.where(iota == j, r, lr)
        lrank_v[pl.ds(v * LANES, LANES)] = lr
        return 0

    lax.fori_loop(0, RVECS, pass1, 0)

    cnt = jnp.zeros((LANES,), jnp.int32)
    for e in range(N_EXP):
        cnt = jnp.where(iota == e, run_s[e], cnt)
    cnt_v[...] = cnt
    pltpu.sync_copy(cnt_v, cnt_hbm.at[wid])
    pltpu.sync_copy(lrank_v, lrank_hbm.at[pl.ds(base, TPT)])


def _offset_body(idx_hbm, lrank_hbm, cnt_hbm, rank_hbm, off_hbm, idx_v,
                 lrank_v, all_v, off_v, woff_s):
    wid = _wid()
    base = pl.multiple_of(wid * TPT, 8)
    pltpu.sync_copy(cnt_hbm, all_v)
    pltpu.sync_copy(idx_hbm.at[pl.ds(base, TPT)], idx_v)
    pltpu.sync_copy(lrank_hbm.at[pl.ds(base, TPT)], lrank_v)
    iota = lax.broadcasted_iota(jnp.int32, (LANES,), 0)

    before = jnp.zeros((LANES,), jnp.int32)
    tot = jnp.zeros((LANES,), jnp.int32)
    for w in range(NW):
        row = all_v[w]
        before = jnp.where(w < wid, before + row, before)
        tot = tot + row

    offs = jnp.full((LANES,), N_TOK, jnp.int32)
    base_e = jnp.zeros((), jnp.int32)
    for e in range(N_EXP):
        woff_s[e] = base_e + before[e]
        offs = jnp.where(iota == e, base_e, offs)
        base_e = base_e + tot[e]

    def pass2(v, _):
        ev = idx_v[pl.ds(v * LANES, LANES)]
        lr = lrank_v[pl.ds(v * LANES, LANES)]
        add = jnp.zeros((LANES,), jnp.int32)
        for j in range(LANES):
            add = jnp.where(iota == j, woff_s[ev[j]], add)
        lrank_v[pl.ds(v * LANES, LANES)] = lr + add
        return 0

    lax.fori_loop(0, RVECS, pass2, 0)
    pltpu.sync_copy(lrank_v, rank_hbm.at[pl.ds(base, TPT)])

    @pl.when(wid == 0)
    def _():
        off_v[...] = offs
        pltpu.sync_copy(off_v, off_hbm)


def _routing(idx32):
    mesh = plsc.VectorSubcoreMesh(core_axis_name="c", subcore_axis_name="s")
    count = pl.kernel(
        _count_body,
        out_type=(jax.ShapeDtypeStruct((N_TOK,), jnp.int32),
                  jax.ShapeDtypeStruct((NW, LANES), jnp.int32)),
        mesh=mesh,
        scratch_types=[
            pltpu.VMEM((TPT,), jnp.int32),
            pltpu.VMEM((TPT,), jnp.int32),
            pltpu.VMEM((LANES,), jnp.int32),
            pltpu.SMEM((N_EXP,), jnp.int32),
        ],
    )
    lrank, cnt = count(idx32)
    offset = pl.kernel(
        _offset_body,
        out_type=(jax.ShapeDtypeStruct((N_TOK,), jnp.int32),
                  jax.ShapeDtypeStruct((LANES,), jnp.int32)),
        mesh=mesh,
        scratch_types=[
            pltpu.VMEM((TPT,), jnp.int32),
            pltpu.VMEM((TPT,), jnp.int32),
            pltpu.VMEM((NW, LANES), jnp.int32),
            pltpu.VMEM((LANES,), jnp.int32),
            pltpu.SMEM((N_EXP,), jnp.int32),
        ],
    )
    return offset(idx32, lrank, cnt)


def _scatter_body(x_hbm, rank3_hbm, xs_hbm, idx_v, rows_v, sem):
    wid = _wid()
    pltpu.sync_copy(rank3_hbm.at[wid], idx_v)
    base = pl.multiple_of(wid * RPW, 8)
    for ch in range(NCHUNK):
        pltpu.sync_copy(x_hbm.at[pl.ds(base + ch * CHUNK, CHUNK)], rows_v)
        pltpu.async_copy(rows_v, xs_hbm.at[idx_v.at[ch]], sem).wait()


def _scatter_rows(x, rank3):
    mesh = plsc.VectorSubcoreMesh(core_axis_name="c", subcore_axis_name="s")
    f = pl.kernel(
        _scatter_body,
        out_type=jax.ShapeDtypeStruct((N_TOK, D_IN), jnp.float32),
        mesh=mesh,
        scratch_types=[
            pltpu.VMEM((NCHUNK, CHUNK), jnp.int32),
            pltpu.VMEM((CHUNK, D_IN), jnp.float32),
            pltpu.SemaphoreType.DMA,
        ],
    )
    return f(x, rank3)


def _gather_body(ys_hbm, rank3_hbm, y_hbm, idx_v, rows_v, sem):
    wid = _wid()
    pltpu.sync_copy(rank3_hbm.at[wid], idx_v)
    base = pl.multiple_of(wid * RPW, 8)
    for ch in range(NCHUNK):
        pltpu.async_copy(ys_hbm.at[idx_v.at[ch]], rows_v, sem).wait()
        pltpu.sync_copy(rows_v, y_hbm.at[pl.ds(base + ch * CHUNK, CHUNK)])


def _gather_rows(ys, rank3):
    mesh = plsc.VectorSubcoreMesh(core_axis_name="c", subcore_axis_name="s")
    f = pl.kernel(
        _gather_body,
        out_type=jax.ShapeDtypeStruct((N_TOK, D_OUT), jnp.float32),
        mesh=mesh,
        scratch_types=[
            pltpu.VMEM((NCHUNK, CHUNK), jnp.int32),
            pltpu.VMEM((CHUNK, D_OUT), jnp.float32),
            pltpu.SemaphoreType.DMA,
        ],
    )
    return f(ys, rank3)


def _span(off_ref, t):
    r_lo = t * TBLK
    r_hi = r_lo + TBLK - 1
    e_lo = jnp.zeros((), jnp.int32)
    e_hi = jnp.zeros((), jnp.int32)
    for i in range(N_EXP - 1):
        e_lo = e_lo + (off_ref[i + 1] <= r_lo).astype(jnp.int32)
        e_hi = e_hi + (off_ref[i + 1] <= r_hi).astype(jnp.int32)
    return e_lo, e_hi


def _w_index(t, k, off_ref):
    e_lo, e_hi = _span(off_ref, t)
    return (jnp.minimum(e_lo + k, e_hi), 0, 0)


def _mm_body(off_ref, x_ref, w_ref, b_ref, o_ref, xbf_ref):
    t = pl.program_id(0)
    k = pl.program_id(1)
    e_lo, e_hi = _span(off_ref, t)
    e = jnp.minimum(e_lo + k, e_hi)

    @pl.when(k == 0)
    def _():
        xbf_ref[...] = x_ref[...].astype(jnp.bfloat16)

    @pl.when(k <= e_hi - e_lo)
    def _():
        rows = t * TBLK + lax.broadcasted_iota(jnp.int32, (TBLK, 1), 0)
        member = ((rows >= off_ref[e]) & (rows < off_ref[e + 1])
                  ).astype(jnp.float32)
        xm = xbf_ref[...] * member.astype(jnp.bfloat16)
        mm = jnp.dot(xm, w_ref[0], preferred_element_type=jnp.float32)
        contrib = mm + member * b_ref[0]

        @pl.when(k == 0)
        def _():
            o_ref[...] = contrib

        @pl.when(k > 0)
        def _():
            o_ref[...] += contrib


def _grouped_matmul(xs, W_bf, b3, offsets):
    grid_spec = pltpu.PrefetchScalarGridSpec(
        num_scalar_prefetch=1,
        grid=(T_BLOCKS, KMAX),
        in_specs=[
            pl.BlockSpec((TBLK, D_IN), lambda t, k, off: (t, 0)),
            pl.BlockSpec((1, D_IN, D_OUT), _w_index),
            pl.BlockSpec((1, 1, D_OUT),
                         lambda t, k, off: (_w_index(t, k, off)[0], 0, 0)),
        ],
        out_specs=pl.BlockSpec((TBLK, D_OUT), lambda t, k, off: (t, 0)),
        scratch_shapes=[pltpu.VMEM((TBLK, D_IN), jnp.bfloat16)],
    )
    return pl.pallas_call(
        _mm_body,
        grid_spec=grid_spec,
        out_shape=jax.ShapeDtypeStruct((N_TOK, D_OUT), jnp.float32),
    )(offsets, xs, W_bf, b3)


def kernel(x, layer_idx, W, b):
    idx32 = layer_idx.astype(jnp.int32)
    rank, offsets = _routing(idx32)
    rank3 = rank.reshape(NW, NCHUNK, CHUNK)
    xs = _scatter_rows(x, rank3)
    ys = _grouped_matmul(xs, W.astype(jnp.bfloat16),
                         b.reshape(N_EXP, 1, D_OUT), offsets)
    return _gather_rows(ys, rank3)


# speedup vs baseline: 532.1363x; 1.0213x over previous
import functools

import jax
import jax.numpy as jnp
from jax import lax
from jax.experimental import pallas as pl
from jax.experimental.pallas import tpu as pltpu
from jax.experimental.pallas import tpu_sc as plsc

N_TOK = 8192
N_EXP = 8
D_IN = 2048
D_OUT = 2048

NC = 2
NS = 16
NW = NC * NS
LANES = 16

TPT = N_TOK // NW
RVECS = TPT // LANES

RPW = N_TOK // NW
CHUNK = 16
NCHUNK = RPW // CHUNK

TBLK = 512
T_BLOCKS = N_TOK // TBLK
KMAX = N_EXP


def _wid():
    return lax.axis_index("c") * NS + lax.axis_index("s")


def _count_body(idx_hbm, lrank_hbm, cnt_hbm, idx_v, lrank_v, cnt_v, run_s):
    wid = _wid()
    base = pl.multiple_of(wid * TPT, 8)
    pltpu.sync_copy(idx_hbm.at[pl.ds(base, TPT)], idx_v)
    iota = lax.broadcasted_iota(jnp.int32, (LANES,), 0)

    for e in range(N_EXP):
        run_s[e] = 0

    def pass1(v, _):
        ev = idx_v[pl.ds(v * LANES, LANES)]
        lr = jnp.zeros((LANES,), jnp.int32)
        for j in range(LANES):
            e = ev[j]
            r = run_s[e]
            run_s[e] = r + 1
            lr = jnp.where(iota == j, r, lr)
        lrank_v[pl.ds(v * LANES, LANES)] = lr
        return 0

    lax.fori_loop(0, RVECS, pass1, 0)

    cnt = jnp.zeros((LANES,), jnp.int32)
    for e in range(N_EXP):
        cnt = jnp.where(iota == e, run_s[e], cnt)
    cnt_v[...] = cnt
    pltpu.sync_copy(cnt_v, cnt_hbm.at[wid])
    pltpu.sync_copy(lrank_v, lrank_hbm.at[pl.ds(base, TPT)])


def _offset_body(idx_hbm, lrank_hbm, cnt_hbm, rank_hbm, off_hbm, idx_v,
                 lrank_v, all_v, off_v, woff_s):
    wid = _wid()
    base = pl.multiple_of(wid * TPT, 8)
    pltpu.sync_copy(cnt_hbm, all_v)
    pltpu.sync_copy(idx_hbm.at[pl.ds(base, TPT)], idx_v)
    pltpu.sync_copy(lrank_hbm.at[pl.ds(base, TPT)], lrank_v)
    iota = lax.broadcasted_iota(jnp.int32, (LANES,), 0)

    before = jnp.zeros((LANES,), jnp.int32)
    tot = jnp.zeros((LANES,), jnp.int32)
    for w in range(NW):
        row = all_v[w]
        before = jnp.where(w < wid, before + row, before)
        tot = tot + row

    offs = jnp.full((LANES,), N_TOK, jnp.int32)
    base_e = jnp.zeros((), jnp.int32)
    for e in range(N_EXP):
        woff_s[e] = base_e + before[e]
        offs = jnp.where(iota == e, base_e, offs)
        base_e = base_e + tot[e]

    def pass2(v, _):
        ev = idx_v[pl.ds(v * LANES, LANES)]
        lr = lrank_v[pl.ds(v * LANES, LANES)]
        add = jnp.zeros((LANES,), jnp.int32)
        for j in range(LANES):
            add = jnp.where(iota == j, woff_s[ev[j]], add)
        lrank_v[pl.ds(v * LANES, LANES)] = lr + add
        return 0

    lax.fori_loop(0, RVECS, pass2, 0)
    pltpu.sync_copy(lrank_v, rank_hbm.at[pl.ds(base, TPT)])

    @pl.when(wid == 0)
    def _():
        off_v[...] = offs
        pltpu.sync_copy(off_v, off_hbm)


def _routing(idx32):
    mesh = plsc.VectorSubcoreMesh(core_axis_name="c", subcore_axis_name="s")
    count = pl.kernel(
        _count_body,
        out_type=(jax.ShapeDtypeStruct((N_TOK,), jnp.int32),
                  jax.ShapeDtypeStruct((NW, LANES), jnp.int32)),
        mesh=mesh,
        scratch_types=[
            pltpu.VMEM((TPT,), jnp.int32),
            pltpu.VMEM((TPT,), jnp.int32),
            pltpu.VMEM((LANES,), jnp.int32),
            pltpu.SMEM((N_EXP,), jnp.int32),
        ],
    )
    lrank, cnt = count(idx32)
    offset = pl.kernel(
        _offset_body,
        out_type=(jax.ShapeDtypeStruct((N_TOK,), jnp.int32),
                  jax.ShapeDtypeStruct((LANES,), jnp.int32)),
        mesh=mesh,
        scratch_types=[
            pltpu.VMEM((TPT,), jnp.int32),
            pltpu.VMEM((TPT,), jnp.int32),
            pltpu.VMEM((NW, LANES), jnp.int32),
            pltpu.VMEM((LANES,), jnp.int32),
            pltpu.SMEM((N_EXP,), jnp.int32),
        ],
    )
    return offset(idx32, lrank, cnt)


def _scatter_body(x_hbm, rank3_hbm, xs_hbm, idx_v, rows0_v, rows1_v,
                  sem_i0, sem_i1, sem_o0, sem_o1):
    wid = _wid()
    pltpu.sync_copy(rank3_hbm.at[wid], idx_v)
    base = pl.multiple_of(wid * RPW, 8)
    bufs = (rows0_v, rows1_v)
    sem_i = (sem_i0, sem_i1)
    sem_o = (sem_o0, sem_o1)

    def start_in(ch):
        return pltpu.async_copy(
            x_hbm.at[pl.ds(base + ch * CHUNK, CHUNK)], bufs[ch % 2],
            sem_i[ch % 2])

    h_in = {0: start_in(0)}
    h_out = {}
    for ch in range(NCHUNK):
        b = ch % 2
        h_in[ch].wait()
        if ch + 1 < NCHUNK:
            if ch >= 1:
                h_out[ch - 1].wait()
            h_in[ch + 1] = start_in(ch + 1)
        h_out[ch] = pltpu.async_copy(bufs[b], xs_hbm.at[idx_v.at[ch]],
                                     sem_o[b])
    h_out[NCHUNK - 2].wait()
    h_out[NCHUNK - 1].wait()


def _scatter_rows(x, rank3):
    mesh = plsc.VectorSubcoreMesh(core_axis_name="c", subcore_axis_name="s")
    f = pl.kernel(
        _scatter_body,
        out_type=jax.ShapeDtypeStruct((N_TOK, D_IN), jnp.float32),
        mesh=mesh,
        scratch_types=[
            pltpu.VMEM((NCHUNK, CHUNK), jnp.int32),
            pltpu.VMEM((CHUNK, D_IN), jnp.float32),
            pltpu.VMEM((CHUNK, D_IN), jnp.float32),
            pltpu.SemaphoreType.DMA,
            pltpu.SemaphoreType.DMA,
            pltpu.SemaphoreType.DMA,
            pltpu.SemaphoreType.DMA,
        ],
    )
    return f(x, rank3)


def _gather_body(ys_hbm, rank3_hbm, y_hbm, idx_v, rows0_v, rows1_v,
                 sem_i0, sem_i1, sem_o0, sem_o1):
    wid = _wid()
    pltpu.sync_copy(rank3_hbm.at[wid], idx_v)
    base = pl.multiple_of(wid * RPW, 8)
    bufs = (rows0_v, rows1_v)
    sem_i = (sem_i0, sem_i1)
    sem_o = (sem_o0, sem_o1)

    def start_in(ch):
        return pltpu.async_copy(ys_hbm.at[idx_v.at[ch]], bufs[ch % 2],
                                sem_i[ch % 2])

    h_in = {0: start_in(0)}
    h_out = {}
    for ch in range(NCHUNK):
        b = ch % 2
        h_in[ch].wait()
        if ch + 1 < NCHUNK:
            if ch >= 1:
                h_out[ch - 1].wait()
            h_in[ch + 1] = start_in(ch + 1)
        h_out[ch] = pltpu.async_copy(
            bufs[b], y_hbm.at[pl.ds(base + ch * CHUNK, CHUNK)], sem_o[b])
    h_out[NCHUNK - 2].wait()
    h_out[NCHUNK - 1].wait()


def _gather_rows(ys, rank3):
    mesh = plsc.VectorSubcoreMesh(core_axis_name="c", subcore_axis_name="s")
    f = pl.kernel(
        _gather_body,
        out_type=jax.ShapeDtypeStruct((N_TOK, D_OUT), jnp.float32),
        mesh=mesh,
        scratch_types=[
            pltpu.VMEM((NCHUNK, CHUNK), jnp.int32),
            pltpu.VMEM((CHUNK, D_OUT), jnp.float32),
            pltpu.VMEM((CHUNK, D_OUT), jnp.float32),
            pltpu.SemaphoreType.DMA,
            pltpu.SemaphoreType.DMA,
            pltpu.SemaphoreType.DMA,
            pltpu.SemaphoreType.DMA,
        ],
    )
    return f(ys, rank3)


def _span(off_ref, t):
    r_lo = t * TBLK
    r_hi = r_lo + TBLK - 1
    e_lo = jnp.zeros((), jnp.int32)
    e_hi = jnp.zeros((), jnp.int32)
    for i in range(N_EXP - 1):
        e_lo = e_lo + (off_ref[i + 1] <= r_lo).astype(jnp.int32)
        e_hi = e_hi + (off_ref[i + 1] <= r_hi).astype(jnp.int32)
    return e_lo, e_hi


def _w_index(t, k, off_ref):
    e_lo, e_hi = _span(off_ref, t)
    return (jnp.minimum(e_lo + k, e_hi), 0, 0)


def _mm_body(off_ref, x_ref, w_ref, b_ref, o_ref, xbf_ref):
    t = pl.program_id(0)
    k = pl.program_id(1)
    e_lo, e_hi = _span(off_ref, t)
    e = jnp.minimum(e_lo + k, e_hi)

    @pl.when(k == 0)
    def _():
        xbf_ref[...] = x_ref[...].astype(jnp.bfloat16)

    @pl.when(k <= e_hi - e_lo)
    def _():
        rows = t * TBLK + lax.broadcasted_iota(jnp.int32, (TBLK, 1), 0)
        member = ((rows >= off_ref[e]) & (rows < off_ref[e + 1])
                  ).astype(jnp.float32)
        xm = xbf_ref[...] * member.astype(jnp.bfloat16)
        mm = jnp.dot(xm, w_ref[0], preferred_element_type=jnp.float32)
        contrib = mm + member * b_ref[0]

        @pl.when(k == 0)
        def _():
            o_ref[...] = contrib

        @pl.when(k > 0)
        def _():
            o_ref[...] += contrib


def _grouped_matmul(xs, W_bf, b3, offsets):
    grid_spec = pltpu.PrefetchScalarGridSpec(
        num_scalar_prefetch=1,
        grid=(T_BLOCKS, KMAX),
        in_specs=[
            pl.BlockSpec((TBLK, D_IN), lambda t, k, off: (t, 0)),
            pl.BlockSpec((1, D_IN, D_OUT), _w_index),
            pl.BlockSpec((1, 1, D_OUT),
                         lambda t, k, off: (_w_index(t, k, off)[0], 0, 0)),
        ],
        out_specs=pl.BlockSpec((TBLK, D_OUT), lambda t, k, off: (t, 0)),
        scratch_shapes=[pltpu.VMEM((TBLK, D_IN), jnp.bfloat16)],
    )
    return pl.pallas_call(
        _mm_body,
        grid_spec=grid_spec,
        out_shape=jax.ShapeDtypeStruct((N_TOK, D_OUT), jnp.float32),
    )(offsets, xs, W_bf, b3)


def kernel(x, layer_idx, W, b):
    idx32 = layer_idx.astype(jnp.int32)
    rank, offsets = _routing(idx32)
    rank3 = rank.reshape(NW, NCHUNK, CHUNK)
    xs = _scatter_rows(x, rank3)
    ys = _grouped_matmul(xs, W.astype(jnp.bfloat16),
                         b.reshape(N_EXP, 1, D_OUT), offsets)
    return _gather_rows(ys, rank3)


# speedup vs baseline: 643.1717x; 1.2087x over previous
import functools

import jax
import jax.numpy as jnp
from jax import lax
from jax.experimental import pallas as pl
from jax.experimental.pallas import tpu as pltpu
from jax.experimental.pallas import tpu_sc as plsc

N_TOK = 8192
N_EXP = 8
D_IN = 2048
D_OUT = 2048

NC = 2
NS = 16
NW = NC * NS
LANES = 16

TPT = N_TOK // NW
RVECS = TPT // LANES

RPW = N_TOK // NW
CHUNK = 16
NCHUNK = RPW // CHUNK

TBLK = 512
T_BLOCKS = N_TOK // TBLK
S_MAX = 24


def _wid():
    return lax.axis_index("c") * NS + lax.axis_index("s")


def _count_body(idx_hbm, lrank_hbm, cnt_hbm, idx_v, lrank_v, cnt_v, run_s):
    wid = _wid()
    base = pl.multiple_of(wid * TPT, 8)
    pltpu.sync_copy(idx_hbm.at[pl.ds(base, TPT)], idx_v)
    iota = lax.broadcasted_iota(jnp.int32, (LANES,), 0)

    for e in range(N_EXP):
        run_s[e] = 0

    def pass1(v, _):
        ev = idx_v[pl.ds(v * LANES, LANES)]
        lr = jnp.zeros((LANES,), jnp.int32)
        for j in range(LANES):
            e = ev[j]
            r = run_s[e]
            run_s[e] = r + 1
            lr = jnp.where(iota == j, r, lr)
        lrank_v[pl.ds(v * LANES, LANES)] = lr
        return 0

    lax.fori_loop(0, RVECS, pass1, 0)

    cnt = jnp.zeros((LANES,), jnp.int32)
    for e in range(N_EXP):
        cnt = jnp.where(iota == e, run_s[e], cnt)
    cnt_v[...] = cnt
    pltpu.sync_copy(cnt_v, cnt_hbm.at[wid])
    pltpu.sync_copy(lrank_v, lrank_hbm.at[pl.ds(base, TPT)])


def _offset_body(idx_hbm, lrank_hbm, cnt_hbm, rank_hbm, off_hbm, sched_hbm,
                 idx_v, lrank_v, all_v, off_v, sched_v, woff_s):
    wid = _wid()
    base = pl.multiple_of(wid * TPT, 8)
    pltpu.sync_copy(cnt_hbm, all_v)
    pltpu.sync_copy(idx_hbm.at[pl.ds(base, TPT)], idx_v)
    pltpu.sync_copy(lrank_hbm.at[pl.ds(base, TPT)], lrank_v)
    iota = lax.broadcasted_iota(jnp.int32, (LANES,), 0)

    before = jnp.zeros((LANES,), jnp.int32)
    tot = jnp.zeros((LANES,), jnp.int32)
    for w in range(NW):
        row = all_v[w]
        before = jnp.where(w < wid, before + row, before)
        tot = tot + row

    offs = jnp.full((LANES,), N_TOK, jnp.int32)
    base_e = jnp.zeros((), jnp.int32)
    for e in range(N_EXP):
        woff_s[e] = base_e + before[e]
        offs = jnp.where(iota == e, base_e, offs)
        base_e = base_e + tot[e]

    def pass2(v, _):
        ev = idx_v[pl.ds(v * LANES, LANES)]
        lr = lrank_v[pl.ds(v * LANES, LANES)]
        add = jnp.zeros((LANES,), jnp.int32)
        for j in range(LANES):
            add = jnp.where(iota == j, woff_s[ev[j]], add)
        lrank_v[pl.ds(v * LANES, LANES)] = lr + add
        return 0

    lax.fori_loop(0, RVECS, pass2, 0)
    pltpu.sync_copy(lrank_v, rank_hbm.at[pl.ds(base, TPT)])

    @pl.when(wid == 0)
    def _():
        off_v[...] = offs
        pltpu.sync_copy(off_v, off_hbm)

        o = [offs[i] for i in range(N_EXP + 1)]

        def e_lo_of(tq):
            acc = jnp.zeros((), jnp.int32)
            for i in range(N_EXP - 1):
                acc = acc + (o[i + 1] <= tq * TBLK).astype(jnp.int32)
            return acc

        def e_hi_of(tq):
            acc = jnp.zeros((), jnp.int32)
            for i in range(N_EXP - 1):
                acc = acc + (o[i + 1] <= tq * TBLK + TBLK - 1).astype(
                    jnp.int32)
            return acc

        t = jnp.zeros((), jnp.int32)
        e = e_lo_of(t)
        sv0 = jnp.zeros((LANES,), jnp.int32)
        sv1 = jnp.zeros((LANES,), jnp.int32)
        for step in range(S_MAX):
            valid = (t < T_BLOCKS).astype(jnp.int32)
            tc = jnp.minimum(t, T_BLOCKS - 1)
            val = tc + 32 * e + 512 * valid
            if step < LANES:
                sv0 = jnp.where(iota == step, val, sv0)
            else:
                sv1 = jnp.where(iota == (step - LANES), val, sv1)
            adv_e = e < e_hi_of(tc)
            tn = jnp.where(adv_e, t, t + 1)
            e = jnp.where(adv_e, e + 1,
                          e_lo_of(jnp.minimum(tn, T_BLOCKS - 1)))
            t = tn
        sched_v[pl.ds(0, LANES)] = sv0
        sched_v[pl.ds(LANES, LANES)] = sv1
        pltpu.sync_copy(sched_v, sched_hbm)


def _routing(idx32):
    mesh = plsc.VectorSubcoreMesh(core_axis_name="c", subcore_axis_name="s")
    count = pl.kernel(
        _count_body,
        out_type=(jax.ShapeDtypeStruct((N_TOK,), jnp.int32),
                  jax.ShapeDtypeStruct((NW, LANES), jnp.int32)),
        mesh=mesh,
        scratch_types=[
            pltpu.VMEM((TPT,), jnp.int32),
            pltpu.VMEM((TPT,), jnp.int32),
            pltpu.VMEM((LANES,), jnp.int32),
            pltpu.SMEM((N_EXP,), jnp.int32),
        ],
    )
    lrank, cnt = count(idx32)
    offset = pl.kernel(
        _offset_body,
        out_type=(jax.ShapeDtypeStruct((N_TOK,), jnp.int32),
                  jax.ShapeDtypeStruct((LANES,), jnp.int32),
                  jax.ShapeDtypeStruct((2 * LANES,), jnp.int32)),
        mesh=mesh,
        scratch_types=[
            pltpu.VMEM((TPT,), jnp.int32),
            pltpu.VMEM((TPT,), jnp.int32),
            pltpu.VMEM((NW, LANES), jnp.int32),
            pltpu.VMEM((LANES,), jnp.int32),
            pltpu.VMEM((2 * LANES,), jnp.int32),
            pltpu.SMEM((N_EXP,), jnp.int32),
        ],
    )
    return offset(idx32, lrank, cnt)


def _scatter_body(x_hbm, rank3_hbm, xs_hbm, idx_v, rows0_v, rows1_v,
                  sem_i0, sem_i1, sem_o0, sem_o1):
    wid = _wid()
    pltpu.sync_copy(rank3_hbm.at[wid], idx_v)
    base = pl.multiple_of(wid * RPW, 8)
    bufs = (rows0_v, rows1_v)
    sem_i = (sem_i0, sem_i1)
    sem_o = (sem_o0, sem_o1)

    def start_in(ch):
        return pltpu.async_copy(
            x_hbm.at[pl.ds(base + ch * CHUNK, CHUNK)], bufs[ch % 2],
            sem_i[ch % 2])

    h_in = {0: start_in(0)}
    h_out = {}
    for ch in range(NCHUNK):
        b = ch % 2
        h_in[ch].wait()
        if ch + 1 < NCHUNK:
            if ch >= 1:
                h_out[ch - 1].wait()
            h_in[ch + 1] = start_in(ch + 1)
        h_out[ch] = pltpu.async_copy(bufs[b], xs_hbm.at[idx_v.at[ch]],
                                     sem_o[b])
    h_out[NCHUNK - 2].wait()
    h_out[NCHUNK - 1].wait()


def _scatter_rows(x, rank3):
    mesh = plsc.VectorSubcoreMesh(core_axis_name="c", subcore_axis_name="s")
    f = pl.kernel(
        _scatter_body,
        out_type=jax.ShapeDtypeStruct((N_TOK, D_IN), jnp.float32),
        mesh=mesh,
        scratch_types=[
            pltpu.VMEM((NCHUNK, CHUNK), jnp.int32),
            pltpu.VMEM((CHUNK, D_IN), jnp.float32),
            pltpu.VMEM((CHUNK, D_IN), jnp.float32),
            pltpu.SemaphoreType.DMA,
            pltpu.SemaphoreType.DMA,
            pltpu.SemaphoreType.DMA,
            pltpu.SemaphoreType.DMA,
        ],
    )
    return f(x, rank3)


def _gather_body(ys_hbm, rank3_hbm, y_hbm, idx_v, rows0_v, rows1_v,
                 sem_i0, sem_i1, sem_o0, sem_o1):
    wid = _wid()
    pltpu.sync_copy(rank3_hbm.at[wid], idx_v)
    base = pl.multiple_of(wid * RPW, 8)
    bufs = (rows0_v, rows1_v)
    sem_i = (sem_i0, sem_i1)
    sem_o = (sem_o0, sem_o1)

    def start_in(ch):
        return pltpu.async_copy(ys_hbm.at[idx_v.at[ch]], bufs[ch % 2],
                                sem_i[ch % 2])

    h_in = {0: start_in(0)}
    h_out = {}
    for ch in range(NCHUNK):
        b = ch % 2
        h_in[ch].wait()
        if ch + 1 < NCHUNK:
            if ch >= 1:
                h_out[ch - 1].wait()
            h_in[ch + 1] = start_in(ch + 1)
        h_out[ch] = pltpu.async_copy(
            bufs[b], y_hbm.at[pl.ds(base + ch * CHUNK, CHUNK)], sem_o[b])
    h_out[NCHUNK - 2].wait()
    h_out[NCHUNK - 1].wait()


def _gather_rows(ys, rank3):
    mesh = plsc.VectorSubcoreMesh(core_axis_name="c", subcore_axis_name="s")
    f = pl.kernel(
        _gather_body,
        out_type=jax.ShapeDtypeStruct((N_TOK, D_OUT), jnp.float32),
        mesh=mesh,
        scratch_types=[
            pltpu.VMEM((NCHUNK, CHUNK), jnp.int32),
            pltpu.VMEM((CHUNK, D_OUT), jnp.float32),
            pltpu.VMEM((CHUNK, D_OUT), jnp.float32),
            pltpu.SemaphoreType.DMA,
            pltpu.SemaphoreType.DMA,
            pltpu.SemaphoreType.DMA,
            pltpu.SemaphoreType.DMA,
        ],
    )
    return f(ys, rank3)


def _mm_body(off_ref, sched_ref, x_ref, w_ref, b_ref, o_ref, xbf_ref):
    s = pl.program_id(0)
    v = sched_ref[s]
    t = v % 32
    e = (v // 32) % 16
    valid = v // 512
    vprev = sched_ref[jnp.maximum(s - 1, 0)]
    first = (s == 0) | (t != vprev % 32)

    @pl.when(first)
    def _():
        xbf_ref[...] = x_ref[...].astype(jnp.bfloat16)

    @pl.when(valid > 0)
    def _():
        rows = t * TBLK + lax.broadcasted_iota(jnp.int32, (TBLK, 1), 0)
        member = ((rows >= off_ref[e]) & (rows < off_ref[e + 1])
                  ).astype(jnp.float32)
        xm = xbf_ref[...] * member.astype(jnp.bfloat16)
        mm = jnp.dot(xm, w_ref[0], preferred_element_type=jnp.float32)
        contrib = mm + member * b_ref[0]

        @pl.when(first)
        def _():
            o_ref[...] = contrib

        @pl.when(jnp.logical_not(first))
        def _():
            o_ref[...] += contrib


def _grouped_matmul(xs, W_bf, b3, offsets, sched):
    grid_spec = pltpu.PrefetchScalarGridSpec(
        num_scalar_prefetch=2,
        grid=(S_MAX,),
        in_specs=[
            pl.BlockSpec((TBLK, D_IN),
                         lambda s, off, sc: (sc[s] % 32, 0)),
            pl.BlockSpec((1, D_IN, D_OUT),
                         lambda s, off, sc: ((sc[s] // 32) % 16, 0, 0)),
            pl.BlockSpec((1, 1, D_OUT),
                         lambda s, off, sc: ((sc[s] // 32) % 16, 0, 0)),
        ],
        out_specs=pl.BlockSpec((TBLK, D_OUT),
                               lambda s, off, sc: (sc[s] % 32, 0)),
        scratch_shapes=[pltpu.VMEM((TBLK, D_IN), jnp.bfloat16)],
    )
    return pl.pallas_call(
        _mm_body,
        grid_spec=grid_spec,
        out_shape=jax.ShapeDtypeStruct((N_TOK, D_OUT), jnp.float32),
    )(offsets, sched, xs, W_bf, b3)


def kernel(x, layer_idx, W, b):
    idx32 = layer_idx.astype(jnp.int32)
    rank, offsets, sched = _routing(idx32)
    rank3 = rank.reshape(NW, NCHUNK, CHUNK)
    xs = _scatter_rows(x, rank3)
    ys = _grouped_matmul(xs, W.astype(jnp.bfloat16),
                         b.reshape(N_EXP, 1, D_OUT), offsets, sched)
    return _gather_rows(ys, rank3)
